# baseline (speedup 1.0000x reference)
import functools

import jax
import jax.numpy as jnp
from jax import lax
from jax.experimental import pallas as pl
from jax.experimental.pallas import tpu as pltpu
from jax.experimental.pallas import tpu_sc as plsc

BATCH = 16384
EMBED_DIM = 64
HIDDEN_DIM = 256

NUM_CORES = 2
NUM_SUBCORES = 16
NUM_WORKERS = NUM_CORES * NUM_SUBCORES
B_PER_W = BATCH // NUM_WORKERS
CHUNK = 128
CHUNKS_PER_W = B_PER_W // CHUNK

MLP_TILE = 1024


def _gather_body(user_table, item_table, uid2, iid2, u_out, i_out,
                 uidx_v, iidx_v, urows_v, irows_v, usem, isem):
    wid = lax.axis_index("s") * NUM_CORES + lax.axis_index("c")
    base = wid * B_PER_W
    row0 = wid * CHUNKS_PER_W
    pltpu.sync_copy(uid2.at[pl.ds(row0, CHUNKS_PER_W)], uidx_v)
    pltpu.sync_copy(iid2.at[pl.ds(row0, CHUNKS_PER_W)], iidx_v)
    copies = []
    for j in range(CHUNKS_PER_W):
        copies.append(pltpu.async_copy(
            user_table.at[uidx_v.at[j]],
            urows_v.at[pl.ds(j * CHUNK, CHUNK)], usem))
        copies.append(pltpu.async_copy(
            item_table.at[iidx_v.at[j]],
            irows_v.at[pl.ds(j * CHUNK, CHUNK)], isem))
    for c in copies:
        c.wait()
    pltpu.sync_copy(urows_v, u_out.at[pl.ds(base, B_PER_W)])
    pltpu.sync_copy(irows_v, i_out.at[pl.ds(base, B_PER_W)])


def _sc_gather(user_table, item_table, user_id, item_id):
    uid2 = user_id.reshape(BATCH // CHUNK, CHUNK)
    iid2 = item_id.reshape(BATCH // CHUNK, CHUNK)
    emb = jax.ShapeDtypeStruct((BATCH, EMBED_DIM), jnp.float32)
    fn = functools.partial(
        pl.kernel,
        mesh=plsc.VectorSubcoreMesh(core_axis_name="c", subcore_axis_name="s"),
        compiler_params=pltpu.CompilerParams(use_tc_tiling_on_sc=False),
        out_type=(emb, emb),
        scratch_types=[
            pltpu.VMEM((CHUNKS_PER_W, CHUNK), jnp.int32),
            pltpu.VMEM((CHUNKS_PER_W, CHUNK), jnp.int32),
            pltpu.VMEM((B_PER_W, EMBED_DIM), jnp.float32),
            pltpu.VMEM((B_PER_W, EMBED_DIM), jnp.float32),
            pltpu.SemaphoreType.DMA,
            pltpu.SemaphoreType.DMA,
        ],
    )(_gather_body)
    return fn(user_table, item_table, uid2, iid2)


def _mlp_body(u_ref, i_ref, xf_ref, w1u_ref, w1i_ref, w1f_ref, b1_ref,
              w2_ref, b2_ref, o_ref):
    h = jnp.dot(u_ref[...], w1u_ref[...], preferred_element_type=jnp.float32)
    h = h + jnp.dot(i_ref[...], w1i_ref[...], preferred_element_type=jnp.float32)
    h = h + jnp.dot(xf_ref[...], w1f_ref[...], preferred_element_type=jnp.float32)
    h = jnp.maximum(h + b1_ref[...], 0.0)
    y = jnp.dot(h, w2_ref[...], preferred_element_type=jnp.float32) + b2_ref[...]
    o_ref[...] = jax.nn.sigmoid(y)


def _tc_mlp(u_emb, i_emb, xf, W1, b1, W2, b2):
    w1u = W1[:EMBED_DIM]
    w1i = W1[EMBED_DIM:2 * EMBED_DIM]
    w1f = W1[2 * EMBED_DIM:]
    b1_2d = b1.reshape(1, HIDDEN_DIM)
    b2_2d = b2.reshape(1, 1)
    grid = BATCH // MLP_TILE
    out = pl.pallas_call(
        _mlp_body,
        grid=(grid,),
        in_specs=[
            pl.BlockSpec((MLP_TILE, EMBED_DIM), lambda t: (t, 0)),
            pl.BlockSpec((MLP_TILE, EMBED_DIM), lambda t: (t, 0)),
            pl.BlockSpec((MLP_TILE, 2), lambda t: (t, 0)),
            pl.BlockSpec((EMBED_DIM, HIDDEN_DIM), lambda t: (0, 0)),
            pl.BlockSpec((EMBED_DIM, HIDDEN_DIM), lambda t: (0, 0)),
            pl.BlockSpec((2, HIDDEN_DIM), lambda t: (0, 0)),
            pl.BlockSpec((1, HIDDEN_DIM), lambda t: (0, 0)),
            pl.BlockSpec((HIDDEN_DIM, 1), lambda t: (0, 0)),
            pl.BlockSpec((1, 1), lambda t: (0, 0)),
        ],
        out_specs=pl.BlockSpec((MLP_TILE, 1), lambda t: (t, 0)),
        out_shape=jax.ShapeDtypeStruct((BATCH, 1), jnp.float32),
    )(u_emb, i_emb, xf, w1u, w1i, w1f, b1_2d, W2, b2_2d)
    return out[:, 0]


def kernel(user_id, item_id, user_feature, item_feature, user_table,
           item_table, W1, b1, W2, b2):
    u_emb, i_emb = _sc_gather(user_table, item_table, user_id, item_id)
    xf = jnp.stack([user_feature, item_feature], axis=1)
    return _tc_mlp(u_emb, i_emb, xf, W1, b1, W2, b2)
